# baseline (speedup 1.0000x reference)
import functools

import jax
import jax.numpy as jnp
from jax import lax
from jax.experimental import pallas as pl
from jax.experimental.pallas import tpu as pltpu
from jax.experimental.pallas import tpu_sc as plsc

N = 4194304
NT = 16
C = N // NT
W = 16384
NWIN = C // W
BITS = 10
R = 1 << BITS
PASSES = 3
CH = 128
NCH = W // CH
DEPTH = 4

_mesh = plsc.VectorSubcoreMesh(
    core_axis_name="c", subcore_axis_name="s", num_cores=1)


def _i32(*shape):
  return jax.ShapeDtypeStruct(shape, jnp.int32)


def _make_pass(shift):
  @functools.partial(
      pl.kernel,
      out_type=(_i32(N), _i32(N), _i32(NT, R)),
      mesh=_mesh,
      compiler_params=pltpu.CompilerParams(needs_layout_passes=False),
      scratch_types=[
          pltpu.VMEM((W,), jnp.int32),
          pltpu.VMEM((W,), jnp.int32),
          pltpu.VMEM((NCH, CH), jnp.int32),
          pltpu.VMEM((R,), jnp.int32),
          pltpu.VMEM((R,), jnp.int32),
          pltpu.VMEM((NT, R), jnp.int32),
          pltpu.SemaphoreType.DMA,
          pltpu.SemaphoreType.DMA,
      ],
  )
  def one_pass(src_k, src_v, dst_k, dst_v, grid_h,
               kw, vw, idx, hist, offs, grid, sem_k, sem_v):
    sid = lax.axis_index("s")
    base = sid * C
    zero = jnp.zeros((16,), jnp.int32)

    def zero_body(b, _):
      hist[pl.ds(b * 16, 16)] = zero
      return 0
    lax.fori_loop(0, R // 16, zero_body, 0)

    def hist_win(w, _):
      pltpu.sync_copy(src_k.at[pl.ds(base + w * W, W)], kw)

      def hist_chunk(c, _):
        for j in range(CH // 16):
          k = kw[pl.ds(c * CH + j * 16, 16)]
          dig = lax.shift_right_logical(k, shift) & (R - 1)
          cnt, last = plsc.scan_count(dig)
          plsc.addupdate_scatter(hist, [dig], cnt, mask=last)
        return 0
      lax.fori_loop(0, NCH, hist_chunk, 0)
      return 0
    lax.fori_loop(0, NWIN, hist_win, 0)

    pltpu.sync_copy(hist, grid_h.at[sid])
    plsc.subcore_barrier()
    pltpu.sync_copy(grid_h, grid)

    def scan_body(b, carry):
      tot = zero
      pre = zero
      for t in range(NT):
        row = grid[t, pl.ds(b * 16, 16)]
        tot = tot + row
        pre = pre + row * (jnp.int32(t) < sid).astype(jnp.int32)
      inc = plsc.cumsum(tot)
      offs[pl.ds(b * 16, 16)] = (inc - tot) + pre + carry
      return carry + jnp.sum(tot)
    lax.fori_loop(0, R // 16, scan_body, jnp.int32(0))

    def perm_win(w, _):
      pltpu.sync_copy(src_k.at[pl.ds(base + w * W, W)], kw)
      pltpu.sync_copy(src_v.at[pl.ds(base + w * W, W)], vw)

      def perm_chunk(c, _):
        for j in range(CH // 16):
          k = kw[pl.ds(c * CH + j * 16, 16)]
          dig = lax.shift_right_logical(k, shift) & (R - 1)
          cnt, last = plsc.scan_count(dig)
          o = plsc.load_gather(offs, [dig])
          idx[c, pl.ds(j * 16, 16)] = o + cnt - 1
          plsc.addupdate_scatter(offs, [dig], cnt, mask=last)
        pltpu.async_copy(kw.at[pl.ds(c * CH, CH)], dst_k.at[idx.at[c]], sem_k)
        pltpu.async_copy(vw.at[pl.ds(c * CH, CH)], dst_v.at[idx.at[c]], sem_v)

        @pl.when(c >= DEPTH)
        def _():
          pltpu.make_async_copy(
              kw.at[pl.ds(0, CH)], dst_k.at[idx.at[0]], sem_k).wait()
          pltpu.make_async_copy(
              vw.at[pl.ds(0, CH)], dst_v.at[idx.at[0]], sem_v).wait()
        return 0
      lax.fori_loop(0, NCH, perm_chunk, 0)

      for _ in range(DEPTH):
        pltpu.make_async_copy(
            kw.at[pl.ds(0, CH)], dst_k.at[idx.at[0]], sem_k).wait()
        pltpu.make_async_copy(
            vw.at[pl.ds(0, CH)], dst_v.at[idx.at[0]], sem_v).wait()
      return 0
    lax.fori_loop(0, NWIN, perm_win, 0)

  return one_pass


_passes = [_make_pass(BITS * p) for p in range(PASSES)]


@jax.jit
def _sort(keys, values):
  k, v = keys, values
  for one_pass in _passes:
    k, v, _ = one_pass(k, v)
  return k, v


def kernel(keys, values):
  return _sort(keys, values)


# speedup vs baseline: 1.5073x; 1.5073x over previous
import functools

import jax
import jax.numpy as jnp
from jax import lax
from jax.experimental import pallas as pl
from jax.experimental.pallas import tpu as pltpu
from jax.experimental.pallas import tpu_sc as plsc

N = 4194304
NT = 16
C = N // NT
W = 8192
NWIN = C // W
BITS = 15
R = 1 << BITS
PASSES = 2
NQ = 16
QB = R // NQ
CH = 128
NCH = W // CH
DEPTH = 4

_mesh = plsc.VectorSubcoreMesh(
    core_axis_name="c", subcore_axis_name="s", num_cores=1)


def _i32(*shape):
  return jax.ShapeDtypeStruct(shape, jnp.int32)


def _make_pass(shift):
  @functools.partial(
      pl.kernel,
      out_type=(_i32(N), _i32(N), _i32(NQ, NT, QB)),
      mesh=_mesh,
      compiler_params=pltpu.CompilerParams(needs_layout_passes=False),
      scratch_types=[
          pltpu.VMEM((W,), jnp.int32),
          pltpu.VMEM((W,), jnp.int32),
          pltpu.VMEM((NCH, CH), jnp.int32),
          pltpu.VMEM((R,), jnp.int32),
          pltpu.VMEM((R,), jnp.int32),
          pltpu.VMEM((NT, QB), jnp.int32),
          pltpu.SemaphoreType.DMA,
          pltpu.SemaphoreType.DMA,
      ],
  )
  def one_pass(src_k, src_v, dst_k, dst_v, grid_h,
               kw, vw, idx, hist, offs, gchunk, sem_k, sem_v):
    sid = lax.axis_index("s")
    base = sid * C
    zero = jnp.zeros((16,), jnp.int32)

    def zero_body(b, _):
      hist[pl.ds(b * 16, 16)] = zero
      return 0
    lax.fori_loop(0, R // 16, zero_body, 0)

    def hist_win(w, _):
      pltpu.sync_copy(src_k.at[pl.ds(base + w * W, W)], kw)

      def hist_chunk(c, _):
        for j in range(CH // 16):
          k = kw[pl.ds(c * CH + j * 16, 16)]
          dig = lax.shift_right_logical(k, shift) & (R - 1)
          cnt, last = plsc.scan_count(dig)
          plsc.addupdate_scatter(hist, [dig], cnt, mask=last)
        return 0
      lax.fori_loop(0, NCH, hist_chunk, 0)
      return 0
    lax.fori_loop(0, NWIN, hist_win, 0)

    for q in range(NQ):
      pltpu.sync_copy(hist.at[pl.ds(q * QB, QB)], grid_h.at[q, sid])
    plsc.subcore_barrier()

    carry = jnp.int32(0)
    for q in range(NQ):
      pltpu.sync_copy(grid_h.at[q], gchunk)

      def scan_body(b, carry, q=q):
        tot = zero
        pre = zero
        for t in range(NT):
          row = gchunk[t, pl.ds(b * 16, 16)]
          tot = tot + row
          pre = pre + row * (jnp.int32(t) < sid).astype(jnp.int32)
        inc = plsc.cumsum(tot)
        offs[pl.ds(q * QB + b * 16, 16)] = (inc - tot) + pre + carry
        return carry + jnp.sum(tot)
      carry = lax.fori_loop(0, QB // 16, scan_body, carry)

    def perm_win(w, _):
      pltpu.sync_copy(src_k.at[pl.ds(base + w * W, W)], kw)
      pltpu.sync_copy(src_v.at[pl.ds(base + w * W, W)], vw)

      def perm_chunk(c, _):
        for j in range(CH // 16):
          k = kw[pl.ds(c * CH + j * 16, 16)]
          dig = lax.shift_right_logical(k, shift) & (R - 1)
          cnt, last = plsc.scan_count(dig)
          o = plsc.load_gather(offs, [dig])
          idx[c, pl.ds(j * 16, 16)] = o + cnt - 1
          plsc.addupdate_scatter(offs, [dig], cnt, mask=last)
        pltpu.async_copy(kw.at[pl.ds(c * CH, CH)], dst_k.at[idx.at[c]], sem_k)
        pltpu.async_copy(vw.at[pl.ds(c * CH, CH)], dst_v.at[idx.at[c]], sem_v)

        @pl.when(c >= DEPTH)
        def _():
          pltpu.make_async_copy(
              kw.at[pl.ds(0, CH)], dst_k.at[idx.at[0]], sem_k).wait()
          pltpu.make_async_copy(
              vw.at[pl.ds(0, CH)], dst_v.at[idx.at[0]], sem_v).wait()
        return 0
      lax.fori_loop(0, NCH, perm_chunk, 0)

      for _ in range(DEPTH):
        pltpu.make_async_copy(
            kw.at[pl.ds(0, CH)], dst_k.at[idx.at[0]], sem_k).wait()
        pltpu.make_async_copy(
            vw.at[pl.ds(0, CH)], dst_v.at[idx.at[0]], sem_v).wait()
      return 0
    lax.fori_loop(0, NWIN, perm_win, 0)

  return one_pass


_passes = [_make_pass(BITS * p) for p in range(PASSES)]


@jax.jit
def _sort(keys, values):
  k, v = keys, values
  for one_pass in _passes:
    k, v, _ = one_pass(k, v)
  return k, v


def kernel(keys, values):
  return _sort(keys, values)


# speedup vs baseline: 1.5247x; 1.0116x over previous
import functools

import jax
import jax.numpy as jnp
from jax import lax
from jax.experimental import pallas as pl
from jax.experimental.pallas import tpu as pltpu
from jax.experimental.pallas import tpu_sc as plsc

N = 4194304
NC = 2
NS = 16
NW = NC * NS
C = N // NW
W = 8192
NWIN = C // W
BITS = 15
R = 1 << BITS
PASSES = 2
NQ = 16
QB = R // NQ
CH = 128
NCH = W // CH
DEPTH = 4

_mesh = plsc.VectorSubcoreMesh(core_axis_name="c", subcore_axis_name="s")
_cparams = pltpu.CompilerParams(needs_layout_passes=False)


def _i32(*shape):
  return jax.ShapeDtypeStruct(shape, jnp.int32)


def _make_hist(shift):
  @functools.partial(
      pl.kernel,
      out_type=_i32(NQ, NW, QB),
      mesh=_mesh,
      compiler_params=_cparams,
      scratch_types=[
          pltpu.VMEM((W,), jnp.int32),
          pltpu.VMEM((R,), jnp.int32),
      ],
  )
  def hist_kernel(src_k, grid_h, kw, hist):
    wid = lax.axis_index("c") * NS + lax.axis_index("s")
    base = wid * C
    zero = jnp.zeros((16,), jnp.int32)

    def zero_body(b, _):
      hist[pl.ds(b * 16, 16)] = zero
      return 0
    lax.fori_loop(0, R // 16, zero_body, 0)

    def hist_win(w, _):
      pltpu.sync_copy(src_k.at[pl.ds(base + w * W, W)], kw)

      def hist_chunk(c, _):
        for j in range(CH // 16):
          k = kw[pl.ds(c * CH + j * 16, 16)]
          dig = lax.shift_right_logical(k, shift) & (R - 1)
          cnt, last = plsc.scan_count(dig)
          plsc.addupdate_scatter(hist, [dig], cnt, mask=last)
        return 0
      lax.fori_loop(0, NCH, hist_chunk, 0)
      return 0
    lax.fori_loop(0, NWIN, hist_win, 0)

    for q in range(NQ):
      pltpu.sync_copy(hist.at[pl.ds(q * QB, QB)], grid_h.at[q, wid])

  return hist_kernel


def _make_perm(shift):
  @functools.partial(
      pl.kernel,
      out_type=(_i32(N), _i32(N)),
      mesh=_mesh,
      compiler_params=_cparams,
      scratch_types=[
          pltpu.VMEM((W,), jnp.int32),
          pltpu.VMEM((W,), jnp.int32),
          pltpu.VMEM((NCH, CH), jnp.int32),
          pltpu.VMEM((R,), jnp.int32),
          pltpu.VMEM((NW, QB), jnp.int32),
          pltpu.SemaphoreType.DMA,
          pltpu.SemaphoreType.DMA,
      ],
  )
  def perm_kernel(src_k, src_v, grid_h, dst_k, dst_v,
                  kw, vw, idx, offs, gchunk, sem_k, sem_v):
    wid = lax.axis_index("c") * NS + lax.axis_index("s")
    base = wid * C
    zero = jnp.zeros((16,), jnp.int32)

    carry = jnp.int32(0)
    for q in range(NQ):
      pltpu.sync_copy(grid_h.at[q], gchunk)

      def scan_body(b, carry, q=q):
        tot = zero
        pre = zero
        for t in range(NW):
          row = gchunk[t, pl.ds(b * 16, 16)]
          tot = tot + row
          pre = pre + row * (jnp.int32(t) < wid).astype(jnp.int32)
        inc = plsc.cumsum(tot)
        offs[pl.ds(q * QB + b * 16, 16)] = (inc - tot) + pre + carry
        return carry + jnp.sum(tot)
      carry = lax.fori_loop(0, QB // 16, scan_body, carry)

    def perm_win(w, _):
      pltpu.sync_copy(src_k.at[pl.ds(base + w * W, W)], kw)
      pltpu.sync_copy(src_v.at[pl.ds(base + w * W, W)], vw)

      def perm_chunk(c, _):
        for j in range(CH // 16):
          k = kw[pl.ds(c * CH + j * 16, 16)]
          dig = lax.shift_right_logical(k, shift) & (R - 1)
          cnt, last = plsc.scan_count(dig)
          o = plsc.load_gather(offs, [dig])
          idx[c, pl.ds(j * 16, 16)] = o + cnt - 1
          plsc.addupdate_scatter(offs, [dig], cnt, mask=last)
        pltpu.async_copy(kw.at[pl.ds(c * CH, CH)], dst_k.at[idx.at[c]], sem_k)
        pltpu.async_copy(vw.at[pl.ds(c * CH, CH)], dst_v.at[idx.at[c]], sem_v)

        @pl.when(c >= DEPTH)
        def _():
          pltpu.make_async_copy(
              kw.at[pl.ds(0, CH)], dst_k.at[idx.at[0]], sem_k).wait()
          pltpu.make_async_copy(
              vw.at[pl.ds(0, CH)], dst_v.at[idx.at[0]], sem_v).wait()
        return 0
      lax.fori_loop(0, NCH, perm_chunk, 0)

      for _ in range(DEPTH):
        pltpu.make_async_copy(
            kw.at[pl.ds(0, CH)], dst_k.at[idx.at[0]], sem_k).wait()
        pltpu.make_async_copy(
            vw.at[pl.ds(0, CH)], dst_v.at[idx.at[0]], sem_v).wait()
      return 0
    lax.fori_loop(0, NWIN, perm_win, 0)

  return perm_kernel


_stages = [(_make_hist(BITS * p), _make_perm(BITS * p)) for p in range(PASSES)]


@jax.jit
def _sort(keys, values):
  k, v = keys, values
  for hist_kernel, perm_kernel in _stages:
    grid_h = hist_kernel(k)
    k, v = perm_kernel(k, v, grid_h)
  return k, v


def kernel(keys, values):
  return _sort(keys, values)
